# baseline (speedup 1.0000x reference)
import functools

import jax
import jax.numpy as jnp
from jax import lax
from jax.experimental import pallas as pl
from jax.experimental.pallas import tpu as pltpu
from jax.experimental.pallas import tpu_sc as plsc

_MASK_RATIO = 0.15
_LANES = 16


def _make_mask_kernel(seq_len: int, n_idx_pad: int):
    mesh = plsc.VectorSubcoreMesh(core_axis_name="c", subcore_axis_name="s")
    n_workers = 32
    chunk = seq_len // n_workers

    @functools.partial(
        pl.kernel,
        out_type=jax.ShapeDtypeStruct((seq_len,), jnp.float32),
        mesh=mesh,
        scratch_types=[
            pltpu.VMEM((n_idx_pad,), jnp.int32),
            pltpu.VMEM((chunk,), jnp.float32),
            pltpu.SemaphoreType.DMA,
        ],
        compiler_params=pltpu.CompilerParams(needs_layout_passes=False),
    )
    def mask_kernel(idx_hbm, mask_hbm, idx_v, mask_v, idx_sem):
        wid = lax.axis_index("s") * 2 + lax.axis_index("c")
        base = wid * chunk

        idx_cp = pltpu.make_async_copy(idx_hbm, idx_v, idx_sem)
        idx_cp.start()

        ones = jnp.ones((_LANES,), jnp.float32)
        for i in range(chunk // _LANES):
            mask_v[pl.ds(i * _LANES, _LANES)] = ones

        idx_cp.wait()
        zeros = jnp.zeros((_LANES,), jnp.float32)

        for j in range(n_idx_pad // _LANES):
            idx = idx_v[pl.ds(j * _LANES, _LANES)]
            local = idx - base
            in_range = (local >= 0) & (local < chunk)
            local = jnp.clip(local, 0, chunk - 1)
            plsc.store_scatter(mask_v, [local], zeros, mask=in_range)

        pltpu.sync_copy(mask_v, mask_hbm.at[pl.ds(base, chunk)])

    return mask_kernel


def _mul_body(mask_ref, x_ref, o_ref):
    o_ref[...] = x_ref[...] * mask_ref[...]


def _apply_mask(x2d, tmask, blk_rows: int):
    rows, seq_len = x2d.shape
    return pl.pallas_call(
        _mul_body,
        grid=(rows // blk_rows,),
        in_specs=[
            pl.BlockSpec((1, seq_len), lambda i: (0, 0)),
            pl.BlockSpec((blk_rows, seq_len), lambda i: (i, 0)),
        ],
        out_specs=pl.BlockSpec((blk_rows, seq_len), lambda i: (i, 0)),
        out_shape=jax.ShapeDtypeStruct((rows, seq_len), jnp.float32),
    )(tmask, x2d)


def _make_manual_mul(rows: int, seq_len: int, chunk_rows: int, nbuf: int,
                     interpret: bool = False):
    n_chunks = rows // chunk_rows

    def body(mask_hbm, x_hbm, o_hbm, mask_v, inb, outb, mask_sem,
             in_sems, out_sems):
        pltpu.make_async_copy(mask_hbm, mask_v, mask_sem).start()

        def in_start(chunk_i, slot):
            pltpu.make_async_copy(
                x_hbm.at[pl.ds(chunk_i * chunk_rows, chunk_rows), :],
                inb.at[slot],
                in_sems.at[slot],
            ).start()

        for i in range(nbuf):
            in_start(i, i)
        pltpu.make_async_copy(mask_hbm, mask_v, mask_sem).wait()

        def step(i, carry):
            slot = lax.rem(i, nbuf)
            pltpu.make_async_copy(
                x_hbm.at[pl.ds(i * chunk_rows, chunk_rows), :],
                inb.at[slot],
                in_sems.at[slot],
            ).wait()

            @pl.when(i >= nbuf)
            def _():
                pltpu.make_async_copy(
                    o_hbm.at[pl.ds((i - nbuf) * chunk_rows, chunk_rows), :],
                    outb.at[slot],
                    out_sems.at[slot],
                ).wait()

            outb[slot] = inb[slot] * mask_v[...]

            pltpu.make_async_copy(
                outb.at[slot],
                o_hbm.at[pl.ds(i * chunk_rows, chunk_rows), :],
                out_sems.at[slot],
            ).start()

            @pl.when(i + nbuf < n_chunks)
            def _():
                in_start(i + nbuf, slot)

            return carry

        lax.fori_loop(0, n_chunks, step, 0)

        def drain(i, carry):
            slot = lax.rem(i, nbuf)
            pltpu.make_async_copy(
                o_hbm.at[pl.ds(i * chunk_rows, chunk_rows), :],
                outb.at[slot],
                out_sems.at[slot],
            ).wait()
            return carry

        lax.fori_loop(n_chunks - nbuf, n_chunks, drain, 0)

    return pl.pallas_call(
        body,
        in_specs=[
            pl.BlockSpec(memory_space=pl.ANY),
            pl.BlockSpec(memory_space=pl.ANY),
        ],
        out_specs=pl.BlockSpec(memory_space=pl.ANY),
        out_shape=jax.ShapeDtypeStruct((rows, seq_len), jnp.float32),
        scratch_shapes=[
            pltpu.VMEM((1, seq_len), jnp.float32),
            pltpu.VMEM((nbuf, chunk_rows, seq_len), jnp.float32),
            pltpu.VMEM((nbuf, chunk_rows, seq_len), jnp.float32),
            pltpu.SemaphoreType.DMA,
            pltpu.SemaphoreType.DMA((nbuf,)),
            pltpu.SemaphoreType.DMA((nbuf,)),
        ],
        interpret=interpret,
    )


def kernel(x):
    batch, chans, seq_len = x.shape
    n_mask = int(seq_len * _MASK_RATIO)
    perm = jax.random.permutation(jax.random.key(42), seq_len)
    idx = perm[:n_mask].astype(jnp.int32)
    pad = (-n_mask) % _LANES
    idx = jnp.concatenate([idx, jnp.broadcast_to(idx[:1], (pad,))])

    tmask = _make_mask_kernel(seq_len, idx.shape[0])(idx)

    x2d = x.reshape(batch * chans, seq_len)
    out = _apply_mask(x2d, tmask.reshape(1, seq_len), blk_rows=256)
    return out.reshape(batch, chans, seq_len)


# speedup vs baseline: 1.0025x; 1.0025x over previous
import functools

import jax
import jax.numpy as jnp
from jax import lax
from jax.experimental import pallas as pl
from jax.experimental.pallas import tpu as pltpu
from jax.experimental.pallas import tpu_sc as plsc

_MASK_RATIO = 0.15
_LANES = 16


def _make_mask_kernel(seq_len: int, n_idx_pad: int):
    mesh = plsc.VectorSubcoreMesh(core_axis_name="c", subcore_axis_name="s")
    n_workers = 32
    chunk = seq_len // n_workers

    @functools.partial(
        pl.kernel,
        out_type=jax.ShapeDtypeStruct((seq_len,), jnp.float32),
        mesh=mesh,
        scratch_types=[
            pltpu.VMEM((n_idx_pad,), jnp.int32),
            pltpu.VMEM((chunk,), jnp.float32),
            pltpu.SemaphoreType.DMA,
        ],
        compiler_params=pltpu.CompilerParams(needs_layout_passes=False),
    )
    def mask_kernel(idx_hbm, mask_hbm, idx_v, mask_v, idx_sem):
        wid = lax.axis_index("s") * 2 + lax.axis_index("c")
        base = wid * chunk

        idx_cp = pltpu.make_async_copy(idx_hbm, idx_v, idx_sem)
        idx_cp.start()

        ones = jnp.ones((_LANES,), jnp.float32)
        for i in range(chunk // _LANES):
            mask_v[pl.ds(i * _LANES, _LANES)] = ones

        idx_cp.wait()
        zeros = jnp.zeros((_LANES,), jnp.float32)

        def scat(j, carry):
            idx = idx_v[pl.ds(j * _LANES, _LANES)]
            local = idx - base
            in_range = (local >= 0) & (local < chunk)
            local = jnp.clip(local, 0, chunk - 1)
            plsc.store_scatter(mask_v, [local], zeros, mask=in_range)
            return carry

        lax.fori_loop(0, n_idx_pad // _LANES, scat, 0)

        pltpu.sync_copy(mask_v, mask_hbm.at[pl.ds(base, chunk)])

    return mask_kernel


def _mul_body(mask_ref, x_ref, o_ref):
    o_ref[...] = x_ref[...] * mask_ref[...]


def _apply_mask(x2d, tmask, blk_rows: int):
    rows, seq_len = x2d.shape
    return pl.pallas_call(
        _mul_body,
        grid=(rows // blk_rows,),
        in_specs=[
            pl.BlockSpec((1, seq_len), lambda i: (0, 0)),
            pl.BlockSpec((blk_rows, seq_len), lambda i: (i, 0)),
        ],
        out_specs=pl.BlockSpec((blk_rows, seq_len), lambda i: (i, 0)),
        out_shape=jax.ShapeDtypeStruct((rows, seq_len), jnp.float32),
    )(tmask, x2d)


def _make_manual_mul(rows: int, seq_len: int, chunk_rows: int, nbuf: int,
                     interpret: bool = False):
    n_chunks = rows // chunk_rows

    def body(mask_hbm, x_hbm, o_hbm, mask_v, inb, outb, mask_sem,
             in_sems, out_sems):
        pltpu.make_async_copy(mask_hbm, mask_v, mask_sem).start()

        def in_start(chunk_i, slot):
            pltpu.make_async_copy(
                x_hbm.at[pl.ds(chunk_i * chunk_rows, chunk_rows), :],
                inb.at[slot],
                in_sems.at[slot],
            ).start()

        for i in range(nbuf):
            in_start(i, i)
        pltpu.make_async_copy(mask_hbm, mask_v, mask_sem).wait()

        def step(i, carry):
            slot = lax.rem(i, nbuf)
            pltpu.make_async_copy(
                x_hbm.at[pl.ds(i * chunk_rows, chunk_rows), :],
                inb.at[slot],
                in_sems.at[slot],
            ).wait()

            @pl.when(i >= nbuf)
            def _():
                pltpu.make_async_copy(
                    o_hbm.at[pl.ds((i - nbuf) * chunk_rows, chunk_rows), :],
                    outb.at[slot],
                    out_sems.at[slot],
                ).wait()

            outb[slot] = inb[slot] * mask_v[...]

            pltpu.make_async_copy(
                outb.at[slot],
                o_hbm.at[pl.ds(i * chunk_rows, chunk_rows), :],
                out_sems.at[slot],
            ).start()

            @pl.when(i + nbuf < n_chunks)
            def _():
                in_start(i + nbuf, slot)

            return carry

        lax.fori_loop(0, n_chunks, step, 0)

        def drain(i, carry):
            slot = lax.rem(i, nbuf)
            pltpu.make_async_copy(
                o_hbm.at[pl.ds(i * chunk_rows, chunk_rows), :],
                outb.at[slot],
                out_sems.at[slot],
            ).wait()
            return carry

        lax.fori_loop(n_chunks - nbuf, n_chunks, drain, 0)

    return pl.pallas_call(
        body,
        in_specs=[
            pl.BlockSpec(memory_space=pl.ANY),
            pl.BlockSpec(memory_space=pl.ANY),
        ],
        out_specs=pl.BlockSpec(memory_space=pl.ANY),
        out_shape=jax.ShapeDtypeStruct((rows, seq_len), jnp.float32),
        scratch_shapes=[
            pltpu.VMEM((1, seq_len), jnp.float32),
            pltpu.VMEM((nbuf, chunk_rows, seq_len), jnp.float32),
            pltpu.VMEM((nbuf, chunk_rows, seq_len), jnp.float32),
            pltpu.SemaphoreType.DMA,
            pltpu.SemaphoreType.DMA((nbuf,)),
            pltpu.SemaphoreType.DMA((nbuf,)),
        ],
        interpret=interpret,
    )


def kernel(x):
    batch, chans, seq_len = x.shape
    n_mask = int(seq_len * _MASK_RATIO)
    perm = jax.random.permutation(jax.random.key(42), seq_len)
    idx = perm[:n_mask].astype(jnp.int32)
    pad = (-n_mask) % _LANES
    idx = jnp.concatenate([idx, jnp.broadcast_to(idx[:1], (pad,))])

    tmask = _make_mask_kernel(seq_len, idx.shape[0])(idx)

    x2d = x.reshape(batch * chans, seq_len)
    out = _apply_mask(x2d, tmask.reshape(1, seq_len), blk_rows=256)
    return out.reshape(batch, chans, seq_len)
